# baseline (speedup 1.0000x reference)
import functools

import jax
import jax.numpy as jnp
from jax import lax
from jax.experimental import pallas as pl
from jax.experimental.pallas import tpu as pltpu
from jax.experimental.pallas import tpu_sc as plsc

_VHI, _VLO = 8, 128
_NPAD = _VHI * _VLO
_K = 600
_CHUNK = 96
_NEG = -1e30


def _sc_info():
    info = plsc.get_sparse_core_info()
    return info.num_cores, info.num_cores * info.num_subcores


def _sc_max(conf_flat, bs, n, nc):
    ncores, nw = _sc_info()
    bpw = bs // nw
    row_words = n * nc
    ngrp = (n + 15) // 16
    mesh = plsc.VectorSubcoreMesh(core_axis_name="c", subcore_axis_name="s")

    @functools.partial(
        pl.kernel,
        out_type=jax.ShapeDtypeStruct((bs * _NPAD,), jnp.float32),
        mesh=mesh,
        compiler_params=pltpu.CompilerParams(needs_layout_passes=False),
        scratch_types=[
            pltpu.VMEM((row_words,), jnp.float32),
            pltpu.VMEM((_NPAD,), jnp.float32),
        ],
    )
    def k(conf_hbm, m_out, span, mbuf):
        wid = lax.axis_index("s") * ncores + lax.axis_index("c")
        lane = lax.iota(jnp.int32, 16)
        neg = jnp.full((16,), _NEG, jnp.float32)
        for bl in range(bpw):
            b = wid * bpw + bl
            pltpu.sync_copy(conf_hbm.at[pl.ds(b * row_words, row_words)], span)

            def grp(t, _):
                rows = t * 16 + lane
                base_i = rows * nc
                cap = row_words - 1
                m = plsc.load_gather(span, [jnp.minimum(base_i, cap)])
                for c in range(1, nc):
                    v = plsc.load_gather(span, [jnp.minimum(base_i + c, cap)])
                    m = jnp.maximum(m, v)
                m = jnp.where(rows < n, m, neg)
                mbuf[pl.ds(t * 16, 16)] = m
                return ()

            lax.fori_loop(0, ngrp, grp, ())
            for t in range(ngrp, _NPAD // 16):
                mbuf[pl.ds(t * 16, 16)] = neg
            pltpu.sync_copy(mbuf, m_out.at[pl.ds(b * _NPAD, _NPAD)])

    return k(conf_flat)


def _topk_body(m_ref, conf_out_ref, idx_out_ref, *, n):
    bs = m_ref.shape[0]
    key = m_ref[...]

    shape = (bs, _VHI, _VLO)
    v = (lax.broadcasted_iota(jnp.int32, shape, 1) * _VLO
         + lax.broadcasted_iota(jnp.int32, shape, 2))
    b = lax.broadcasted_iota(jnp.int32, shape, 0)
    payload = b * n + v

    kk = 2
    while kk <= _NPAD:
        s = kk // 2
        while s >= 1:
            if s < _VLO:
                ax, sh = 2, s
            else:
                ax, sh = 1, s // _VLO
            upper = (v & s) != 0
            pk = jnp.where(upper, jnp.roll(key, sh, axis=ax),
                           jnp.roll(key, -sh, axis=ax))
            pi = jnp.where(upper, jnp.roll(payload, sh, axis=ax),
                           jnp.roll(payload, -sh, axis=ax))
            precedes = (key > pk) | ((key == pk) & (payload < pi))
            want_small = jnp.logical_not(
                jnp.logical_xor((v & kk) == 0, jnp.logical_not(upper)))
            take = jnp.logical_not(jnp.logical_xor(precedes, want_small))
            key = jnp.where(take, key, pk)
            payload = jnp.where(take, payload, pi)
            s //= 2
        kk *= 2

    conf_out_ref[...] = 1.0 / (1.0 + jnp.exp(-key))
    idx_out_ref[...] = payload


def _tc_topk(m3, n):
    bs = m3.shape[0]
    return pl.pallas_call(
        functools.partial(_topk_body, n=n),
        out_shape=[
            jax.ShapeDtypeStruct((bs, _VHI, _VLO), jnp.float32),
            jax.ShapeDtypeStruct((bs, _VHI, _VLO), jnp.int32),
        ],
    )(m3)


def _sc_gather(feat_flat, anc_flat, idx_flat, total_rows, n, d, ad):
    ncores, nw = _sc_info()
    rows_per_w = total_rows // nw
    nch = rows_per_w // _CHUNK
    assert rows_per_w % _CHUNK == 0
    batches_per_w = (total_rows // _K) // nw
    span_rows = batches_per_w * n
    mesh = plsc.VectorSubcoreMesh(core_axis_name="c", subcore_axis_name="s")

    @functools.partial(
        pl.kernel,
        out_type=[
            jax.ShapeDtypeStruct((total_rows, d), jnp.float32),
            jax.ShapeDtypeStruct((total_rows * ad,), jnp.float32),
        ],
        mesh=mesh,
        compiler_params=pltpu.CompilerParams(needs_layout_passes=False),
        scratch_types=[
            pltpu.VMEM((rows_per_w,), jnp.int32),
            pltpu.VMEM((2, _CHUNK, d), jnp.float32),
            pltpu.VMEM((2 * _CHUNK * ad,), jnp.float32),
            pltpu.VMEM((span_rows * ad + 16,), jnp.float32),
            pltpu.SemaphoreType.DMA,
        ],
    )
    def k(feat_hbm, anc_hbm, idx_hbm, feat_out, anc_out,
          idx_v, fbuf, abuf, anc_span, fsem):
        wid = lax.axis_index("s") * ncores + lax.axis_index("c")
        base = wid * rows_per_w
        span_base = wid * span_rows
        lane = lax.iota(jnp.int32, 16)
        amask = lane < ad
        pltpu.sync_copy(idx_hbm.at[pl.ds(base, rows_per_w)], idx_v)
        pltpu.sync_copy(
            anc_hbm.at[pl.ds(span_base * ad, span_rows * ad)],
            anc_span.at[pl.ds(0, span_rows * ad)])
        for g in range(nch):
            slot = g % 2
            fcp = pltpu.async_copy(
                feat_hbm.at[idx_v.at[pl.ds(g * _CHUNK, _CHUNK)]],
                fbuf.at[slot], fsem)

            def group_body(t, _):
                ivec = idx_v[pl.ds(g * _CHUNK + t * 16, 16)] - span_base
                for l in range(16):
                    rj = ivec[l]
                    vals = plsc.load_gather(anc_span, [rj * ad + lane])
                    plsc.store_scatter(
                        abuf, [(slot * _CHUNK + t * 16 + l) * ad + lane],
                        vals, mask=amask)
                return ()

            lax.fori_loop(0, _CHUNK // 16, group_body, ())
            fcp.wait()
            pltpu.sync_copy(fbuf.at[slot],
                            feat_out.at[pl.ds(base + g * _CHUNK, _CHUNK)])
            pltpu.sync_copy(
                abuf.at[pl.ds(slot * _CHUNK * ad, _CHUNK * ad)],
                anc_out.at[pl.ds((base + g * _CHUNK) * ad, _CHUNK * ad)])

    return k(feat_flat, anc_flat, idx_flat)


def kernel(instance_feature, anchor, confidence):
    bs, n, d = instance_feature.shape
    ad = anchor.shape[-1]
    nc = confidence.shape[-1]

    m_flat = _sc_max(confidence.reshape(bs * n * nc), bs, n, nc)
    conf_sorted, flat_sorted = _tc_topk(m_flat.reshape(bs, _VHI, _VLO), n)

    top_conf = conf_sorted.reshape(bs, _NPAD)[:, :_K]
    flat_idx = flat_sorted.reshape(bs, _NPAD)[:, :_K].reshape(-1)

    feat_flat = instance_feature.reshape(bs * n, d)
    anc_flat = anchor.reshape(bs * n * ad)

    feat_out, anc_out = _sc_gather(feat_flat, anc_flat, flat_idx, bs * _K, n,
                                   d, ad)
    return (top_conf,
            feat_out.reshape(bs, _K, d),
            anc_out.reshape(bs, _K, ad))


# speedup vs baseline: 1.3871x; 1.3871x over previous
import functools

import jax
import jax.numpy as jnp
from jax import lax
from jax.experimental import pallas as pl
from jax.experimental.pallas import tpu as pltpu
from jax.experimental.pallas import tpu_sc as plsc

_VHI, _VLO = 8, 128
_NPAD = _VHI * _VLO
_K = 600
_CHUNK = 96
_NEG = -1e30


def _sc_info():
    info = plsc.get_sparse_core_info()
    return info.num_cores, info.num_cores * info.num_subcores


def _sc_max(conf_flat, bs, n, nc):
    ncores, nw = _sc_info()
    bpw = bs // nw
    row_words = n * nc
    ngrp = (n + 15) // 16
    mesh = plsc.VectorSubcoreMesh(core_axis_name="c", subcore_axis_name="s")

    @functools.partial(
        pl.kernel,
        out_type=jax.ShapeDtypeStruct((bs * _NPAD,), jnp.float32),
        mesh=mesh,
        compiler_params=pltpu.CompilerParams(needs_layout_passes=False),
        scratch_types=[
            pltpu.VMEM((row_words,), jnp.float32),
            pltpu.VMEM((_NPAD,), jnp.float32),
        ],
    )
    def k(conf_hbm, m_out, span, mbuf):
        wid = lax.axis_index("s") * ncores + lax.axis_index("c")
        lane = lax.iota(jnp.int32, 16)
        neg = jnp.full((16,), _NEG, jnp.float32)
        for bl in range(bpw):
            b = wid * bpw + bl
            pltpu.sync_copy(conf_hbm.at[pl.ds(b * row_words, row_words)], span)

            def grp(t, _):
                rows = t * 16 + lane
                base_i = rows * nc
                cap = row_words - 1
                m = plsc.load_gather(span, [jnp.minimum(base_i, cap)])
                for c in range(1, nc):
                    v = plsc.load_gather(span, [jnp.minimum(base_i + c, cap)])
                    m = jnp.maximum(m, v)
                m = jnp.where(rows < n, m, neg)
                mbuf[pl.ds(t * 16, 16)] = m
                return ()

            lax.fori_loop(0, ngrp, grp, ())
            for t in range(ngrp, _NPAD // 16):
                mbuf[pl.ds(t * 16, 16)] = neg
            pltpu.sync_copy(mbuf, m_out.at[pl.ds(b * _NPAD, _NPAD)])

    return k(conf_flat)


def _topk_body(m_ref, conf_out_ref, idx_out_ref, *, n):
    bs = m_ref.shape[0]
    key = m_ref[...]

    del n
    shape = (bs, _VHI, _VLO)
    v = (lax.broadcasted_iota(jnp.int32, shape, 1) * _VLO
         + lax.broadcasted_iota(jnp.int32, shape, 2))
    b = lax.broadcasted_iota(jnp.int32, shape, 0)
    payload = v * bs + b

    kk = 2
    while kk <= _NPAD:
        s = kk // 2
        while s >= 1:
            if s < _VLO:
                ax, sh = 2, s
            else:
                ax, sh = 1, s // _VLO
            upper = (v & s) != 0
            pk = jnp.where(upper, jnp.roll(key, sh, axis=ax),
                           jnp.roll(key, -sh, axis=ax))
            pi = jnp.where(upper, jnp.roll(payload, sh, axis=ax),
                           jnp.roll(payload, -sh, axis=ax))
            precedes = (key > pk) | ((key == pk) & (payload < pi))
            want_small = jnp.logical_not(
                jnp.logical_xor((v & kk) == 0, jnp.logical_not(upper)))
            take = jnp.logical_not(jnp.logical_xor(precedes, want_small))
            key = jnp.where(take, key, pk)
            payload = jnp.where(take, payload, pi)
            s //= 2
        kk *= 2

    conf_out_ref[...] = 1.0 / (1.0 + jnp.exp(-key))
    idx_out_ref[...] = payload


def _tc_topk(m3, n):
    bs = m3.shape[0]
    return pl.pallas_call(
        functools.partial(_topk_body, n=n),
        out_shape=[
            jax.ShapeDtypeStruct((bs, _VHI, _VLO), jnp.float32),
            jax.ShapeDtypeStruct((bs, _VHI, _VLO), jnp.int32),
        ],
    )(m3)


def _sc_gather(feat_flat, anc_flat, idx_flat, total_rows, n, d, ad, bs):
    ncores, nw = _sc_info()
    rows_per_w = total_rows // nw
    nch = rows_per_w // _CHUNK
    assert rows_per_w % _CHUNK == 0
    batches_per_w = (total_rows // _K) // nw
    span_rows = batches_per_w * n
    bs_bits = bs.bit_length() - 1
    assert bs == 1 << bs_bits
    mesh = plsc.VectorSubcoreMesh(core_axis_name="c", subcore_axis_name="s")

    @functools.partial(
        pl.kernel,
        out_type=[
            jax.ShapeDtypeStruct((total_rows, d), jnp.float32),
            jax.ShapeDtypeStruct((total_rows * ad,), jnp.float32),
        ],
        mesh=mesh,
        compiler_params=pltpu.CompilerParams(needs_layout_passes=False),
        scratch_types=[
            pltpu.VMEM((rows_per_w,), jnp.int32),
            pltpu.VMEM((2, _CHUNK, d), jnp.float32),
            pltpu.VMEM((2 * _CHUNK * ad,), jnp.float32),
            pltpu.VMEM((span_rows * ad + 16,), jnp.float32),
            pltpu.SemaphoreType.DMA,
        ],
    )
    def k(feat_hbm, anc_hbm, idx_hbm, feat_out, anc_out,
          idx_v, fbuf, abuf, anc_span, fsem):
        wid = lax.axis_index("s") * ncores + lax.axis_index("c")
        base = wid * rows_per_w
        span_base = wid * span_rows
        lane = lax.iota(jnp.int32, 16)
        amask = lane < ad
        pltpu.sync_copy(idx_hbm.at[pl.ds(base, rows_per_w)], idx_v)
        pltpu.sync_copy(
            anc_hbm.at[pl.ds(span_base * ad, span_rows * ad)],
            anc_span.at[pl.ds(0, span_rows * ad)])
        for g in range(nch):
            slot = g % 2
            fcp = pltpu.async_copy(
                feat_hbm.at[idx_v.at[pl.ds(g * _CHUNK, _CHUNK)]],
                fbuf.at[slot], fsem)

            def group_body(t, _):
                pvec = idx_v[pl.ds(g * _CHUNK + t * 16, 16)]
                ivec = ((pvec & (bs - 1)) - wid * batches_per_w) * n \
                    + (pvec >> bs_bits)
                for l in range(16):
                    rj = ivec[l]
                    vals = plsc.load_gather(anc_span, [rj * ad + lane])
                    plsc.store_scatter(
                        abuf, [(slot * _CHUNK + t * 16 + l) * ad + lane],
                        vals, mask=amask)
                return ()

            lax.fori_loop(0, _CHUNK // 16, group_body, ())
            fcp.wait()
            pltpu.sync_copy(fbuf.at[slot],
                            feat_out.at[pl.ds(base + g * _CHUNK, _CHUNK)])
            pltpu.sync_copy(
                abuf.at[pl.ds(slot * _CHUNK * ad, _CHUNK * ad)],
                anc_out.at[pl.ds((base + g * _CHUNK) * ad, _CHUNK * ad)])

    return k(feat_flat, anc_flat, idx_flat)


def kernel(instance_feature, anchor, confidence):
    bs, n, d = instance_feature.shape
    ad = anchor.shape[-1]
    nc = confidence.shape[-1]

    m_flat = _sc_max(confidence.reshape(bs * n * nc), bs, n, nc)
    conf_sorted, flat_sorted = _tc_topk(m_flat.reshape(bs, _VHI, _VLO), n)

    top_conf = conf_sorted.reshape(bs, _NPAD)[:, :_K]
    flat_idx = flat_sorted.reshape(bs, _NPAD)[:, :_K].reshape(-1)

    feat_flat = jnp.transpose(instance_feature, (1, 0, 2)).reshape(n * bs, d)
    anc_flat = anchor.reshape(bs * n * ad)

    feat_out, anc_out = _sc_gather(feat_flat, anc_flat, flat_idx, bs * _K, n,
                                   d, ad, bs)
    return (top_conf,
            feat_out.reshape(bs, _K, d),
            anc_out.reshape(bs, _K, ad))


# speedup vs baseline: 2.8902x; 2.0836x over previous
import functools

import jax
import jax.numpy as jnp
from jax import lax
from jax.experimental import pallas as pl
from jax.experimental.pallas import tpu as pltpu
from jax.experimental.pallas import tpu_sc as plsc

_NPAD = 1024
_K = 600
_CHUNK = 96
_NEG = -1e30
_ACH = 3200


def _sc_info():
    info = plsc.get_sparse_core_info()
    return info.num_cores, info.num_cores * info.num_subcores


def _topk_body(conf_ref, conf_out_ref, idx_out_ref, *, n, bs):
    nc = conf_ref.shape[0]
    m = conf_ref[0]
    for c in range(1, nc):
        m = jnp.maximum(m, conf_ref[c])
    key = jnp.concatenate(
        [m, jnp.full((_NPAD - n, bs), _NEG, jnp.float32)], axis=0)

    shape = (_NPAD, bs)
    v = lax.broadcasted_iota(jnp.int32, shape, 0)
    b = lax.broadcasted_iota(jnp.int32, shape, 1)
    payload = v * bs + b

    kk = 2
    while kk <= _NPAD:
        s = kk // 2
        while s >= 1:
            upper = (v & s) != 0
            pk = jnp.where(upper, jnp.roll(key, s, axis=0),
                           jnp.roll(key, -s, axis=0))
            pi = jnp.where(upper, jnp.roll(payload, s, axis=0),
                           jnp.roll(payload, -s, axis=0))
            precedes = (key > pk) | ((key == pk) & (payload < pi))
            want_small = jnp.logical_not(
                jnp.logical_xor((v & kk) == 0, jnp.logical_not(upper)))
            take = jnp.logical_not(jnp.logical_xor(precedes, want_small))
            key = jnp.where(take, key, pk)
            payload = jnp.where(take, payload, pi)
            s //= 2
        kk *= 2

    conf_out_ref[...] = 1.0 / (1.0 + jnp.exp(-key))
    idx_out_ref[...] = payload


def _tc_topk(conf_t, n, bs):
    return pl.pallas_call(
        functools.partial(_topk_body, n=n, bs=bs),
        out_shape=[
            jax.ShapeDtypeStruct((_NPAD, bs), jnp.float32),
            jax.ShapeDtypeStruct((_NPAD, bs), jnp.int32),
        ],
    )(conf_t)


def _sc_feat(feat_flat, idx_flat, total_rows, d, bs):
    ncores, nw = _sc_info()
    rows_per_w = total_rows // nw
    nch = rows_per_w // _CHUNK
    bpw = bs // nw
    kpp = 120
    npiece = _K // kpp
    mesh = plsc.VectorSubcoreMesh(core_axis_name="c", subcore_axis_name="s")

    @functools.partial(
        pl.kernel,
        out_type=jax.ShapeDtypeStruct((total_rows, d), jnp.float32),
        mesh=mesh,
        compiler_params=pltpu.CompilerParams(needs_layout_passes=False),
        scratch_types=[
            pltpu.VMEM((kpp * bs,), jnp.int32),
            pltpu.VMEM((rows_per_w,), jnp.int32),
            pltpu.VMEM((2, _CHUNK, d), jnp.float32),
            pltpu.SemaphoreType.DMA,
            pltpu.SemaphoreType.DMA,
            pltpu.SemaphoreType.DMA,
            pltpu.SemaphoreType.DMA,
        ],
    )
    def k(feat_hbm, idx_hbm, feat_out, pbuf, idx_v, fbuf, g0, g1, w0, w1):
        wid = lax.axis_index("s") * ncores + lax.axis_index("c")
        base = wid * rows_per_w
        lane = lax.iota(jnp.int32, 16)
        for piece in range(npiece):
            pltpu.sync_copy(idx_hbm.at[pl.ds(piece * kpp * bs, kpp * bs)],
                            pbuf)
            for bl in range(bpw):
                b_g = wid * bpw + bl

                def ext(t, _, bl=bl, b_g=b_g, piece=piece):
                    kvec = t * 16 + lane
                    msk = kvec < kpp
                    vals = plsc.load_gather(
                        pbuf, [jnp.minimum(kvec, kpp - 1) * bs + b_g])
                    plsc.store_scatter(
                        idx_v, [bl * _K + piece * kpp + kvec], vals, mask=msk)
                    return ()

                lax.fori_loop(0, (kpp + 15) // 16, ext, ())

        gsem = [g0, g1]
        wsem = [w0, w1]

        def gather(g, slot):
            return pltpu.async_copy(
                feat_hbm.at[idx_v.at[pl.ds(g * _CHUNK, _CHUNK)]],
                fbuf.at[slot], gsem[slot])

        wcp = [None, None]
        gcp = [None, None]
        gcp[0] = gather(0, 0)
        for g in range(nch):
            slot = g % 2
            nslot = (g + 1) % 2
            if g + 1 < nch:
                if wcp[nslot] is not None:
                    wcp[nslot].wait()
                gcp[nslot] = gather(g + 1, nslot)
            gcp[slot].wait()
            wcp[slot] = pltpu.async_copy(
                fbuf.at[slot],
                feat_out.at[pl.ds(base + g * _CHUNK, _CHUNK)], wsem[slot])
        wcp[0].wait()
        wcp[1].wait()

    return k(feat_flat, idx_flat)


def _sc_anchor(anc_flat, idx_flat, total_rows, n, ad, bs):
    ncores, nw = _sc_info()
    plane_words = n * bs
    nthird = 3
    jspan = total_rows // nthird
    nach = jspan // _ACH
    mesh = plsc.VectorSubcoreMesh(core_axis_name="c", subcore_axis_name="s")

    @functools.partial(
        pl.kernel,
        out_type=jax.ShapeDtypeStruct((total_rows * ad,), jnp.float32),
        mesh=mesh,
        compiler_params=pltpu.CompilerParams(needs_layout_passes=False),
        scratch_types=[
            pltpu.VMEM((plane_words,), jnp.float32),
            pltpu.VMEM((_ACH,), jnp.int32),
            pltpu.VMEM((_ACH,), jnp.float32),
        ],
    )
    def k(anc_hbm, idx_hbm, anc_out, planebuf, ibuf, obuf):
        wid = lax.axis_index("s") * ncores + lax.axis_index("c")

        def do_item(item):
            plane = item // nthird
            third = item % nthird
            pltpu.sync_copy(
                anc_hbm.at[pl.ds(plane * plane_words, plane_words)], planebuf)
            j0 = third * jspan
            for cch in range(nach):
                pltpu.sync_copy(
                    idx_hbm.at[pl.ds(j0 + cch * _ACH, _ACH)], ibuf)

                def gb(t, _):
                    iv = ibuf[pl.ds(t * 16, 16)]
                    obuf[pl.ds(t * 16, 16)] = plsc.load_gather(planebuf, [iv])
                    return ()

                lax.fori_loop(0, _ACH // 16, gb, ())
                pltpu.sync_copy(
                    obuf,
                    anc_out.at[pl.ds(plane * total_rows + j0 + cch * _ACH,
                                     _ACH)])

        do_item(wid)

        @pl.when(wid == 0)
        def _():
            do_item(nw)

    return k(anc_flat, idx_flat)


def kernel(instance_feature, anchor, confidence):
    bs, n, d = instance_feature.shape
    ad = anchor.shape[-1]

    conf_t = jnp.transpose(confidence, (2, 1, 0))
    feat_flat = jnp.transpose(instance_feature, (1, 0, 2)).reshape(n * bs, d)
    anc_flat = jnp.transpose(anchor, (2, 1, 0)).reshape(ad * n * bs)

    conf_sorted, payload = _tc_topk(conf_t, n, bs)
    top_conf = conf_sorted[:_K].T
    idx_flat = payload.reshape(_NPAD * bs)

    total = _K * bs
    feat_out = _sc_feat(feat_flat, idx_flat, total, d, bs)
    anc_out = _sc_anchor(anc_flat, idx_flat, total, n, ad, bs)

    return (top_conf,
            feat_out.reshape(bs, _K, d),
            jnp.transpose(anc_out.reshape(ad, _K, bs), (2, 1, 0)))


# speedup vs baseline: 3.1361x; 1.0851x over previous
import functools

import jax
import jax.numpy as jnp
from jax import lax
from jax.experimental import pallas as pl
from jax.experimental.pallas import tpu as pltpu
from jax.experimental.pallas import tpu_sc as plsc

_NPAD = 1024
_K = 600
_CHUNK = 120
_NEG = -1e30
_ACH = 3200


def _sc_info():
    info = plsc.get_sparse_core_info()
    return info.num_cores, info.num_cores * info.num_subcores


def _topk_body(conf_ref, conf_out_ref, idx_out_ref, *, n, bs):
    nc = conf_ref.shape[0]
    m = conf_ref[0]
    for c in range(1, nc):
        m = jnp.maximum(m, conf_ref[c])
    key = jnp.concatenate(
        [m, jnp.full((_NPAD - n, bs), _NEG, jnp.float32)], axis=0)

    shape = (_NPAD, bs)
    v = lax.broadcasted_iota(jnp.int32, shape, 0)
    b = lax.broadcasted_iota(jnp.int32, shape, 1)
    payload = v * bs + b

    kk = 2
    while kk <= _NPAD:
        s = kk // 2
        while s >= 1:
            upper = (v & s) != 0
            pk = jnp.where(upper, jnp.roll(key, s, axis=0),
                           jnp.roll(key, -s, axis=0))
            pi = jnp.where(upper, jnp.roll(payload, s, axis=0),
                           jnp.roll(payload, -s, axis=0))
            precedes = (key > pk) | ((key == pk) & (payload < pi))
            want_small = jnp.logical_not(
                jnp.logical_xor((v & kk) == 0, jnp.logical_not(upper)))
            take = jnp.logical_not(jnp.logical_xor(precedes, want_small))
            key = jnp.where(take, key, pk)
            payload = jnp.where(take, payload, pi)
            s //= 2
        kk *= 2

    conf_out_ref[...] = 1.0 / (1.0 + jnp.exp(-key))
    idx_out_ref[...] = payload


def _tc_topk(conf_t, n, bs):
    return pl.pallas_call(
        functools.partial(_topk_body, n=n, bs=bs),
        out_shape=[
            jax.ShapeDtypeStruct((_NPAD, bs), jnp.float32),
            jax.ShapeDtypeStruct((_NPAD, bs), jnp.int32),
        ],
    )(conf_t)


def _sc_feat(feat_flat, idx_flat, total_rows, d, bs):
    ncores, nw = _sc_info()
    rows_per_w = total_rows // nw
    nch = rows_per_w // _CHUNK
    bpw = bs // nw
    kpp = 120
    npiece = _K // kpp
    mesh = plsc.VectorSubcoreMesh(core_axis_name="c", subcore_axis_name="s")

    @functools.partial(
        pl.kernel,
        out_type=jax.ShapeDtypeStruct((total_rows, d), jnp.float32),
        mesh=mesh,
        compiler_params=pltpu.CompilerParams(needs_layout_passes=False),
        scratch_types=[
            pltpu.VMEM((kpp * bs,), jnp.int32),
            pltpu.VMEM((rows_per_w,), jnp.int32),
            pltpu.VMEM((2, _CHUNK, d), jnp.float32),
            pltpu.SemaphoreType.DMA,
            pltpu.SemaphoreType.DMA,
            pltpu.SemaphoreType.DMA,
            pltpu.SemaphoreType.DMA,
        ],
    )
    def k(feat_hbm, idx_hbm, feat_out, pbuf, idx_v, fbuf, g0, g1, w0, w1):
        wid = lax.axis_index("s") * ncores + lax.axis_index("c")
        base = wid * rows_per_w
        lane = lax.iota(jnp.int32, 16)
        for piece in range(npiece):
            pltpu.sync_copy(idx_hbm.at[pl.ds(piece * kpp * bs, kpp * bs)],
                            pbuf)
            for bl in range(bpw):
                b_g = wid * bpw + bl

                def ext(t, _, bl=bl, b_g=b_g, piece=piece):
                    kvec = t * 16 + lane
                    msk = kvec < kpp
                    vals = plsc.load_gather(
                        pbuf, [jnp.minimum(kvec, kpp - 1) * bs + b_g])
                    plsc.store_scatter(
                        idx_v, [bl * _K + piece * kpp + kvec], vals, mask=msk)
                    return ()

                lax.fori_loop(0, (kpp + 15) // 16, ext, ())

        gsem = [g0, g1]
        wsem = [w0, w1]

        def gather(g, slot):
            return pltpu.async_copy(
                feat_hbm.at[idx_v.at[pl.ds(g * _CHUNK, _CHUNK)]],
                fbuf.at[slot], gsem[slot])

        wcp = [None, None]
        gcp = [None, None]
        gcp[0] = gather(0, 0)
        for g in range(nch):
            slot = g % 2
            nslot = (g + 1) % 2
            if g + 1 < nch:
                if wcp[nslot] is not None:
                    wcp[nslot].wait()
                gcp[nslot] = gather(g + 1, nslot)
            gcp[slot].wait()
            wcp[slot] = pltpu.async_copy(
                fbuf.at[slot],
                feat_out.at[pl.ds(base + g * _CHUNK, _CHUNK)], wsem[slot])
        wcp[0].wait()
        wcp[1].wait()

    return k(feat_flat, idx_flat)


def _sc_anchor(anc_flat, idx_flat, total_rows, n, ad, bs):
    ncores, nw = _sc_info()
    plane_words = n * bs
    nthird = 3
    jspan = total_rows // nthird
    nach = jspan // _ACH
    mesh = plsc.VectorSubcoreMesh(core_axis_name="c", subcore_axis_name="s")

    @functools.partial(
        pl.kernel,
        out_type=jax.ShapeDtypeStruct((total_rows * ad,), jnp.float32),
        mesh=mesh,
        compiler_params=pltpu.CompilerParams(needs_layout_passes=False),
        scratch_types=[
            pltpu.VMEM((plane_words,), jnp.float32),
            pltpu.VMEM((_ACH,), jnp.int32),
            pltpu.VMEM((_ACH,), jnp.float32),
        ],
    )
    def k(anc_hbm, idx_hbm, anc_out, planebuf, ibuf, obuf):
        wid = lax.axis_index("s") * ncores + lax.axis_index("c")

        def span_work(plane, j0, nchunks):
            pltpu.sync_copy(
                anc_hbm.at[pl.ds(plane * plane_words, plane_words)], planebuf)
            for cch in range(nchunks):
                pltpu.sync_copy(
                    idx_hbm.at[pl.ds(j0 + cch * _ACH, _ACH)], ibuf)

                def gb(t, _):
                    iv = ibuf[pl.ds(t * 16, 16)]
                    obuf[pl.ds(t * 16, 16)] = plsc.load_gather(planebuf, [iv])
                    return ()

                lax.fori_loop(0, _ACH // 16, gb, ())
                pltpu.sync_copy(
                    obuf,
                    anc_out.at[pl.ds(plane * total_rows + j0 + cch * _ACH,
                                     _ACH)])

        @pl.when(wid < (ad - 1) * nthird)
        def _():
            span_work(wid // nthird, (wid % nthird) * jspan, nach)

        @pl.when(wid >= (ad - 1) * nthird)
        def _():
            half = wid - (ad - 1) * nthird
            span_work(ad - 1, half * (total_rows // 2),
                      total_rows // 2 // _ACH)

    return k(anc_flat, idx_flat)


def kernel(instance_feature, anchor, confidence):
    bs, n, d = instance_feature.shape
    ad = anchor.shape[-1]

    conf_t = jnp.transpose(confidence, (2, 1, 0))
    feat_flat = jnp.transpose(instance_feature, (1, 0, 2)).reshape(n * bs, d)
    anc_flat = jnp.transpose(anchor, (2, 1, 0)).reshape(ad * n * bs)

    conf_sorted, payload = _tc_topk(conf_t, n, bs)
    top_conf = conf_sorted[:_K].T
    idx_flat = payload.reshape(_NPAD * bs)

    total = _K * bs
    feat_out = _sc_feat(feat_flat, idx_flat, total, d, bs)
    anc_out = _sc_anchor(anc_flat, idx_flat, total, n, ad, bs)

    return (top_conf,
            feat_out.reshape(bs, _K, d),
            jnp.transpose(anc_out.reshape(ad, _K, bs), (2, 1, 0)))


# speedup vs baseline: 3.3535x; 1.0693x over previous
import functools

import jax
import jax.numpy as jnp
from jax import lax
from jax.experimental import pallas as pl
from jax.experimental.pallas import tpu as pltpu
from jax.experimental.pallas import tpu_sc as plsc

_NPAD = 1024
_K = 600
_CHUNK = 120
_NEG = -1e30
_ACH = 3200


def _sc_info():
    info = plsc.get_sparse_core_info()
    return info.num_cores, info.num_cores * info.num_subcores


def _topk_body(conf_ref, conf_out_ref, idx_out_ref, *, n, bs):
    nc = conf_ref.shape[0]
    m = conf_ref[0]
    for c in range(1, nc):
        m = jnp.maximum(m, conf_ref[c])
    key = jnp.concatenate(
        [m, jnp.full((_NPAD - n, bs), _NEG, jnp.float32)], axis=0)

    shape = (_NPAD, bs)
    v = lax.broadcasted_iota(jnp.int32, shape, 0)
    b = lax.broadcasted_iota(jnp.int32, shape, 1)
    payload = v * bs + b

    kk = 2
    while kk <= _NPAD:
        s = kk // 2
        while s >= 1:
            upper = (v & s) != 0
            pk = jnp.where(upper, jnp.roll(key, s, axis=0),
                           jnp.roll(key, -s, axis=0))
            pi = jnp.where(upper, jnp.roll(payload, s, axis=0),
                           jnp.roll(payload, -s, axis=0))
            precedes = (key > pk) | ((key == pk) & (payload < pi))
            want_small = jnp.logical_not(
                jnp.logical_xor((v & kk) == 0, jnp.logical_not(upper)))
            take = jnp.logical_not(jnp.logical_xor(precedes, want_small))
            key = jnp.where(take, key, pk)
            payload = jnp.where(take, payload, pi)
            s //= 2
        kk *= 2

    conf_out_ref[...] = 1.0 / (1.0 + jnp.exp(-key))
    idx_out_ref[...] = payload


def _tc_topk(conf_t, n, bs):
    return pl.pallas_call(
        functools.partial(_topk_body, n=n, bs=bs),
        out_shape=[
            jax.ShapeDtypeStruct((_NPAD, bs), jnp.float32),
            jax.ShapeDtypeStruct((_NPAD, bs), jnp.int32),
        ],
    )(conf_t)


def _sc_feat(feat_flat, idx_flat, total_rows, d, bs):
    ncores, nw = _sc_info()
    rows_per_w = total_rows // nw
    nch = rows_per_w // _CHUNK
    bpw = bs // nw
    kpp = 120
    npiece = _K // kpp
    mesh = plsc.VectorSubcoreMesh(core_axis_name="c", subcore_axis_name="s")

    @functools.partial(
        pl.kernel,
        out_type=jax.ShapeDtypeStruct((total_rows, d), jnp.float32),
        mesh=mesh,
        compiler_params=pltpu.CompilerParams(needs_layout_passes=False),
        scratch_types=[
            pltpu.VMEM((kpp * bs,), jnp.int32),
            pltpu.VMEM((rows_per_w,), jnp.int32),
            pltpu.VMEM((2, _CHUNK, d), jnp.float32),
            pltpu.SemaphoreType.DMA,
            pltpu.SemaphoreType.DMA,
            pltpu.SemaphoreType.DMA,
            pltpu.SemaphoreType.DMA,
        ],
    )
    def k(feat_hbm, idx_hbm, feat_out, pbuf, idx_v, fbuf, g0, g1, w0, w1):
        wid = lax.axis_index("s") * ncores + lax.axis_index("c")
        base = wid * rows_per_w
        lane = lax.iota(jnp.int32, 16)
        for piece in range(npiece):
            pltpu.sync_copy(idx_hbm.at[pl.ds(piece * kpp * bs, kpp * bs)],
                            pbuf)
            for bl in range(bpw):
                b_g = wid * bpw + bl

                def ext(t, _, bl=bl, b_g=b_g, piece=piece):
                    kvec = t * 16 + lane
                    msk = kvec < kpp
                    vals = plsc.load_gather(
                        pbuf, [jnp.minimum(kvec, kpp - 1) * bs + b_g])
                    plsc.store_scatter(
                        idx_v, [bl * _K + piece * kpp + kvec], vals, mask=msk)
                    return ()

                lax.fori_loop(0, (kpp + 15) // 16, ext, ())

        gsem = [g0, g1]
        wsem = [w0, w1]

        def gather(g, slot):
            return pltpu.async_copy(
                feat_hbm.at[idx_v.at[pl.ds(g * _CHUNK, _CHUNK)]],
                fbuf.at[slot], gsem[slot])

        wcp = [None, None]
        gcp = [None, None]
        gcp[0] = gather(0, 0)
        for g in range(nch):
            slot = g % 2
            nslot = (g + 1) % 2
            if g + 1 < nch:
                if wcp[nslot] is not None:
                    wcp[nslot].wait()
                gcp[nslot] = gather(g + 1, nslot)
            gcp[slot].wait()
            wcp[slot] = pltpu.async_copy(
                fbuf.at[slot],
                feat_out.at[pl.ds(base + g * _CHUNK, _CHUNK)], wsem[slot])
        wcp[0].wait()
        wcp[1].wait()

    return k(feat_flat, idx_flat)


def _sc_anchor(anc_flat, idx_flat, total_rows, n, ad, bs):
    ncores, nw = _sc_info()
    plane_words = n * bs
    nthird = 3
    jspan = total_rows // nthird
    nach = jspan // _ACH
    mesh = plsc.VectorSubcoreMesh(core_axis_name="c", subcore_axis_name="s")

    @functools.partial(
        pl.kernel,
        out_type=jax.ShapeDtypeStruct((total_rows * ad,), jnp.float32),
        mesh=mesh,
        compiler_params=pltpu.CompilerParams(needs_layout_passes=False),
        scratch_types=[
            pltpu.VMEM((plane_words,), jnp.float32),
            pltpu.VMEM((2 * _ACH,), jnp.int32),
            pltpu.VMEM((2 * _ACH,), jnp.float32),
            pltpu.SemaphoreType.DMA,
            pltpu.SemaphoreType.DMA,
            pltpu.SemaphoreType.DMA,
            pltpu.SemaphoreType.DMA,
            pltpu.SemaphoreType.DMA,
        ],
    )
    def k(anc_hbm, idx_hbm, anc_out, planebuf, ibuf, obuf, psem, i0, i1,
          o0, o1):
        wid = lax.axis_index("s") * ncores + lax.axis_index("c")
        isem = [i0, i1]
        osem = [o0, o1]

        def span_work(plane, j0, nchunks):
            pcp = pltpu.async_copy(
                anc_hbm.at[pl.ds(plane * plane_words, plane_words)], planebuf,
                psem)

            def idx_in(cch, slot):
                return pltpu.async_copy(
                    idx_hbm.at[pl.ds(j0 + cch * _ACH, _ACH)],
                    ibuf.at[pl.ds(slot * _ACH, _ACH)], isem[slot])

            icp = [idx_in(0, 0), None]
            ocp = [None, None]
            pcp.wait()
            for cch in range(nchunks):
                slot = cch % 2
                nslot = (cch + 1) % 2
                if cch + 1 < nchunks:
                    icp[nslot] = idx_in(cch + 1, nslot)
                icp[slot].wait()
                if ocp[slot] is not None:
                    ocp[slot].wait()

                def gb(t, _, slot=slot):
                    iv = ibuf[pl.ds(slot * _ACH + t * 16, 16)]
                    obuf[pl.ds(slot * _ACH + t * 16, 16)] = (
                        plsc.load_gather(planebuf, [iv]))
                    return ()

                lax.fori_loop(0, _ACH // 16, gb, ())
                ocp[slot] = pltpu.async_copy(
                    obuf.at[pl.ds(slot * _ACH, _ACH)],
                    anc_out.at[pl.ds(plane * total_rows + j0 + cch * _ACH,
                                     _ACH)], osem[slot])
            for cp in ocp:
                if cp is not None:
                    cp.wait()

        @pl.when(wid < (ad - 1) * nthird)
        def _():
            span_work(wid // nthird, (wid % nthird) * jspan, nach)

        @pl.when(wid >= (ad - 1) * nthird)
        def _():
            half = wid - (ad - 1) * nthird
            span_work(ad - 1, half * (total_rows // 2),
                      total_rows // 2 // _ACH)

    return k(anc_flat, idx_flat)


def kernel(instance_feature, anchor, confidence):
    bs, n, d = instance_feature.shape
    ad = anchor.shape[-1]

    conf_t = jnp.transpose(confidence, (2, 1, 0))
    feat_flat = jnp.transpose(instance_feature, (1, 0, 2)).reshape(n * bs, d)
    anc_flat = jnp.transpose(anchor, (2, 1, 0)).reshape(ad * n * bs)

    conf_sorted, payload = _tc_topk(conf_t, n, bs)
    top_conf = conf_sorted[:_K].T
    idx_flat = payload.reshape(_NPAD * bs)

    total = _K * bs
    feat_out = _sc_feat(feat_flat, idx_flat, total, d, bs)
    anc_out = _sc_anchor(anc_flat, idx_flat, total, n, ad, bs)

    return (top_conf,
            feat_out.reshape(bs, _K, d),
            jnp.transpose(anc_out.reshape(ad, _K, bs), (2, 1, 0)))
